# Initial kernel scaffold; baseline (speedup 1.0000x reference)
import jax
import jax.numpy as jnp
from jax.experimental import pallas as pl


def kernel(x, Wq, Wk, Wv, Wo):
    raise NotImplementedError("write your pallas kernel here")


# speedup vs baseline: 3.3006x; 3.3006x over previous
import functools

import jax
import jax.numpy as jnp
from jax import lax
from jax.experimental import pallas as pl
from jax.experimental.pallas import tpu as pltpu

H = 16
HD = 64
LR = 0.1
C = 32
F32 = jnp.float32


def _mm_body(x_ref, w_ref, o_ref):
    o_ref[...] = jnp.dot(x_ref[...], w_ref[...], preferred_element_type=F32)


def _matmul(x, w, bm=1024, bn=1024):
    m, k = x.shape
    _, n = w.shape
    return pl.pallas_call(
        _mm_body,
        grid=(m // bm, n // bn),
        in_specs=[
            pl.BlockSpec((bm, k), lambda i, j: (i, 0)),
            pl.BlockSpec((k, bn), lambda i, j: (0, j)),
        ],
        out_specs=pl.BlockSpec((bm, bn), lambda i, j: (i, j)),
        out_shape=jax.ShapeDtypeStruct((m, n), F32),
        compiler_params=pltpu.CompilerParams(
            dimension_semantics=("parallel", "parallel")),
        name="qkv_proj",
    )(x, w)


def _solve_body(k_ref, v_ref, wx_ref, *, r, beta):
    rows = lax.broadcasted_iota(jnp.int32, (r, r), 0)
    cols = lax.broadcasted_iota(jnp.int32, (r, r), 1)
    strict = (cols >> 2) < (rows >> 2)
    ident = jnp.where(rows == cols, 1.0, 0.0)
    for jj in range(2):
        kk = k_ref[:, jj * HD:(jj + 1) * HD]
        vv = v_ref[:, jj * HD:(jj + 1) * HD]
        g = lax.dot_general(kk, kk, (((1,), (1,)), ((), ())),
                            preferred_element_type=F32)
        a = jnp.where(strict, beta * g, 0.0)
        t = ident
        gsz, sh = 4, 2
        while gsz < r:
            u = jnp.dot(t, a, preferred_element_type=F32)
            u = jnp.dot(u, t, preferred_element_type=F32)
            rg = rows >> sh
            cg = cols >> sh
            msub = ((rg & 1) == 1) & (cg == rg - 1)
            t = t - jnp.where(msub, u, 0.0)
            gsz, sh = gsz * 2, sh + 1
        wx_ref[:, jj * 2 * HD:jj * 2 * HD + HD] = (
            jnp.dot(t, vv, preferred_element_type=F32))
        wx_ref[:, jj * 2 * HD + HD:(jj + 1) * 2 * HD] = (
            jnp.dot(t, kk, preferred_element_type=F32))


def _sweep_body(wx_ref, q_ref, k_ref, o_ref, n_ref, *, r, nc, beta, hpc):
    c = pl.program_id(1)

    @pl.when(c == 0)
    def _():
        n_ref[...] = jnp.zeros_like(n_ref)

    rows = lax.broadcasted_iota(jnp.int32, (r, r), 0)
    cols = lax.broadcasted_iota(jnp.int32, (r, r), 1)
    incl = (cols >> 2) <= (rows >> 2)

    for j in range(hpc):
        pr, odd = j >> 1, j & 1
        w = wx_ref[pr * r:(pr + 1) * r, odd * 2 * HD:odd * 2 * HD + HD]
        x = wx_ref[pr * r:(pr + 1) * r, odd * 2 * HD + HD:(odd + 1) * 2 * HD]
        q = q_ref[:, j * HD:(j + 1) * HD]
        kk = k_ref[:, j * HD:(j + 1) * HD]
        n = n_ref[j]
        e = w - jnp.dot(x, n, preferred_element_type=F32)
        qk = lax.dot_general(q, kk, (((1,), (1,)), ((), ())),
                             preferred_element_type=F32)
        aq = jnp.where(incl, beta * qk, 0.0)
        o_ref[:, j * HD:(j + 1) * HD] = (
            jnp.dot(q, n, preferred_element_type=F32)
            + jnp.dot(aq, e, preferred_element_type=F32))
        n_ref[j] = n + beta * lax.dot_general(
            kk, e, (((0,), (0,)), ((), ())),
            preferred_element_type=F32)


def kernel(x, Wq, Wk, Wv, Wo):
    b, s, d = x.shape
    r = C * b
    nc = s // C
    beta = LR / b
    hpc = H // 2

    xt = x.transpose(1, 0, 2).reshape(s * b, d)
    wqkv = jnp.concatenate([Wq.T, Wk.T, Wv.T], axis=1)

    qkv = _matmul(xt, wqkv, bm=1024, bn=1024)

    solve = functools.partial(_solve_body, r=r, beta=beta)
    wx = pl.pallas_call(
        solve,
        grid=(H // 2, nc),
        in_specs=[
            pl.BlockSpec((r, 2 * HD), lambda p, c: (c, H // 2 + p)),
            pl.BlockSpec((r, 2 * HD), lambda p, c: (c, H + p)),
        ],
        out_specs=pl.BlockSpec((r, 4 * HD), lambda p, c: (c * (H // 2) + p, 0)),
        out_shape=jax.ShapeDtypeStruct((nc * (H // 2) * r, 4 * HD), F32),
        compiler_params=pltpu.CompilerParams(
            dimension_semantics=("parallel", "parallel")),
        name="chunk_solve",
    )(qkv, qkv)

    sweep = functools.partial(_sweep_body, r=r, nc=nc, beta=beta, hpc=hpc)
    o = pl.pallas_call(
        sweep,
        grid=(2, nc),
        in_specs=[
            pl.BlockSpec((hpc // 2 * r, 4 * HD), lambda gg, c: (c * 2 + gg, 0)),
            pl.BlockSpec((r, hpc * HD), lambda gg, c: (c, gg)),
            pl.BlockSpec((r, hpc * HD), lambda gg, c: (c, 2 + gg)),
        ],
        out_specs=pl.BlockSpec((r, hpc * HD), lambda gg, c: (c, gg)),
        out_shape=jax.ShapeDtypeStruct((s * b, d), F32),
        scratch_shapes=[pltpu.VMEM((hpc, HD, HD), F32)],
        compiler_params=pltpu.CompilerParams(
            dimension_semantics=("parallel", "arbitrary")),
        name="chunk_sweep",
    )(wx, qkv, qkv)

    out = _matmul(o, Wo.T, bm=1024, bn=1024)
    return out.reshape(s, b, d).transpose(1, 0, 2)
